# Initial kernel scaffold; baseline (speedup 1.0000x reference)
import jax
import jax.numpy as jnp
from jax.experimental import pallas as pl


def kernel(edge_index, edge_index_intents, Gu, Gi):
    raise NotImplementedError("write your pallas kernel here")


# speedup vs baseline: 54.5000x; 54.5000x over previous
import functools

import jax
import jax.numpy as jnp
from jax import lax
from jax.experimental import pallas as pl
from jax.experimental.pallas import tpu as pltpu
from jax.experimental.pallas import tpu_sc as plsc

_N_USERS = 50000
_N_ITEMS = 50000
_N = _N_USERS + _N_ITEMS
_K = 32
_NINT = 4
_E = 1600000

_NC = 2
_NS = 16
_HALF = _N // _NC
_DUMP = _HALF
_ACC_ROWS = _HALF + 48
_ZSTRIPE = _ACC_ROWS // _NS
_B = 128
_EPW = _E // _NS
_NFULL = _EPW // _B
_TAIL = _EPW - _NFULL * _B
_WB = 400
_NCHUNKS = _HALF // _WB
_WROUNDS = (_NCHUNKS + _NS - 1) // _NS
_WBP = 80
_NCHUNKSP = _HALF // _WBP
_WROUNDSP = (_NCHUNKSP + _NS - 1) // _NS

_mesh = plsc.VectorSubcoreMesh(
    core_axis_name="c", subcore_axis_name="s", num_cores=_NC, num_subcores=_NS
)
_params = pltpu.CompilerParams(
    use_tc_tiling_on_sc=False, needs_layout_passes=False
)

_f32 = jnp.float32
_i32 = jnp.int32


def _iota16():
    return lax.iota(_i32, 16)


def _rsqrt_guarded(d):
    s = jnp.maximum(d, 1.0)
    for _ in range(18):
        s = 0.5 * (s + d / s)
    return jnp.where(d > 0.0, 1.0 / s, 0.0)


def _local_idx(raw, c_off):
    v = raw - c_off
    ok = (v >= 0) & (v < _HALF)
    return jnp.where(ok, v, _DUMP)


def _deg_body(row_hbm, col_hbm, int0_hbm, int1_hbm, int2_hbm, int3_hbm, x0_hbm,
              dis1_hbm, dis2_hbm, y0_hbm,
              acc0, acc1, acc2, acc3,
              i4buf, wbuf4, wtail, zbuf,
              irow, icol, irow_l, icol_l,
              irow32, icol32, irow_l32, icol_l32,
              dkbuf, x0buf, d1buf, d2buf, ybuf, tmp64):
    c = lax.axis_index("c")
    s = lax.axis_index("s")
    c_off = c * _HALF
    iota = _iota16()
    int_hbms = (int0_hbm, int1_hbm, int2_hbm, int3_hbm)
    accs = (acc0, acc1, acc2, acc3)

    def _zw(i, carry):
        zbuf[pl.ds(i * 16, 16)] = jnp.zeros((16,), _f32)
        return carry
    lax.fori_loop(0, _B // 16, _zw, 0)

    z0 = s * _ZSTRIPE
    nz_full = _ZSTRIPE // _B
    nz_tail = _ZSTRIPE - nz_full * _B

    def _zacc(j, carry):
        for k in range(_NINT):
            pltpu.sync_copy(zbuf, accs[k].at[pl.ds(z0 + j * _B, _B)])
        return carry
    lax.fori_loop(0, nz_full, _zacc, 0)
    if nz_tail:
        for k in range(_NINT):
            pltpu.sync_copy(zbuf.at[pl.ds(0, nz_tail)],
                            accs[k].at[pl.ds(z0 + nz_full * _B, nz_tail)])
    plsc.subcore_barrier()

    base0 = s * _EPW

    def _softmax(dst, nedges):
        for j in range(nedges // 16):
            sl = pl.ds(j * 16, 16)
            a = [i4buf[k, sl] for k in range(_NINT)]
            m = jnp.maximum(jnp.maximum(a[0], a[1]), jnp.maximum(a[2], a[3]))
            e = [jnp.exp(ak - m) for ak in a]
            ssum = (e[0] + e[1]) + (e[2] + e[3])
            for k in range(_NINT):
                dst[k, sl] = e[k] / ssum

    def _chunk(i, carry):
        base = base0 + i * _B
        pltpu.sync_copy(row_hbm.at[pl.ds(base, _B)], irow)
        pltpu.sync_copy(col_hbm.at[pl.ds(base, _B)], icol)
        for k in range(_NINT):
            pltpu.sync_copy(int_hbms[k].at[pl.ds(base, _B)], i4buf.at[k])
        _softmax(wbuf4, _B)
        for j in range(_B // 16):
            sl = pl.ds(j * 16, 16)
            irow_l[sl] = _local_idx(irow[sl], c_off)
            icol_l[sl] = _local_idx(icol[sl], c_off)
        for k in range(_NINT):
            pltpu.sync_copy(wbuf4.at[k], accs[k].at[irow_l], add=True)
            pltpu.sync_copy(wbuf4.at[k], accs[k].at[icol_l], add=True)
        return carry
    lax.fori_loop(0, _NFULL, _chunk, 0)

    if _TAIL:
        base = base0 + _NFULL * _B
        pltpu.sync_copy(row_hbm.at[pl.ds(base, _TAIL)], irow32)
        pltpu.sync_copy(col_hbm.at[pl.ds(base, _TAIL)], icol32)
        for k in range(_NINT):
            pltpu.sync_copy(int_hbms[k].at[pl.ds(base, _TAIL)],
                            i4buf.at[k, pl.ds(0, _TAIL)])
        _softmax(wtail, _TAIL)
        for j in range(_TAIL // 16):
            sl = pl.ds(j * 16, 16)
            irow_l32[sl] = _local_idx(irow32[sl], c_off)
            icol_l32[sl] = _local_idx(icol32[sl], c_off)
        for k in range(_NINT):
            pltpu.sync_copy(wtail.at[k], accs[k].at[irow_l32], add=True)
            pltpu.sync_copy(wtail.at[k], accs[k].at[icol_l32], add=True)

    plsc.subcore_barrier()

    ibase = lax.shift_left(lax.shift_right_logical(iota, jnp.int32(3)),
                           jnp.int32(4))

    def _wblk(t, carry):
        cid = s + t * _NS

        @pl.when(cid < _NCHUNKS)
        def _():
            r = cid * _WB
            g = c_off + r
            for k in range(_NINT):
                pltpu.sync_copy(accs[k].at[pl.ds(r, _WB)], dkbuf.at[k])
            pltpu.sync_copy(x0_hbm.at[pl.ds(g, _WB)], x0buf)

            def _grp(j, cc):
                sl = pl.ds(j * 16, 16)
                for k in range(_NINT):
                    tmp64[pl.ds(k * 16, 16)] = _rsqrt_guarded(dkbuf[k, sl])
                for t16 in range(16):
                    i = j * 16 + t16
                    g0 = plsc.load_gather(tmp64, [ibase + t16])
                    g1 = plsc.load_gather(tmp64, [ibase + (32 + t16)])
                    lo = pl.ds(0, 16)
                    hi = pl.ds(16, 16)
                    d1buf[i, lo] = g0
                    d1buf[i, hi] = g1
                    d2buf[i, lo] = g0 * g0
                    d2buf[i, hi] = g1 * g1
                    ybuf[i, lo] = x0buf[i, lo] * g0
                    ybuf[i, hi] = x0buf[i, hi] * g1
                return cc
            lax.fori_loop(0, _WB // 16, _grp, 0)

            pltpu.sync_copy(d1buf, dis1_hbm.at[pl.ds(g, _WB)])
            pltpu.sync_copy(d2buf, dis2_hbm.at[pl.ds(g, _WB)])
            pltpu.sync_copy(ybuf, y0_hbm.at[pl.ds(g, _WB)])
        return carry
    lax.fori_loop(0, _WROUNDS, _wblk, 0)


_deg_kernel = functools.partial(
    pl.kernel,
    out_type=(
        jax.ShapeDtypeStruct((_N, _K), _f32),
        jax.ShapeDtypeStruct((_N, _K), _f32),
        jax.ShapeDtypeStruct((_N, _K), _f32),
    ),
    mesh=_mesh,
    compiler_params=_params,
    scratch_types=[
        pltpu.VMEM_SHARED((_ACC_ROWS,), _f32),
        pltpu.VMEM_SHARED((_ACC_ROWS,), _f32),
        pltpu.VMEM_SHARED((_ACC_ROWS,), _f32),
        pltpu.VMEM_SHARED((_ACC_ROWS,), _f32),
        pltpu.VMEM((_NINT, _B), _f32),
        pltpu.VMEM((_NINT, _B), _f32),
        pltpu.VMEM((_NINT, _TAIL), _f32),
        pltpu.VMEM((_B,), _f32),
        pltpu.VMEM((_B,), _i32),
        pltpu.VMEM((_B,), _i32),
        pltpu.VMEM((_B,), _i32),
        pltpu.VMEM((_B,), _i32),
        pltpu.VMEM((_TAIL,), _i32),
        pltpu.VMEM((_TAIL,), _i32),
        pltpu.VMEM((_TAIL,), _i32),
        pltpu.VMEM((_TAIL,), _i32),
        pltpu.VMEM((_NINT, _WB), _f32),
        pltpu.VMEM((_WB, _K), _f32),
        pltpu.VMEM((_WB, _K), _f32),
        pltpu.VMEM((_WB, _K), _f32),
        pltpu.VMEM((_WB, _K), _f32),
        pltpu.VMEM((64,), _f32),
    ],
)(_deg_body)


def _prop_body(y_hbm, scale_hbm, row_hbm, col_hbm, out_hbm,
               acc, irow, icol, icol_l, irow32, icol32, icol_l32,
               gbuf, gbuf32, obuf, sbuf, sem):
    c = lax.axis_index("c")
    s = lax.axis_index("s")
    c_off = c * _HALF

    def _zg(i, carry):
        gbuf[i, pl.ds(0, 16)] = jnp.zeros((16,), _f32)
        gbuf[i, pl.ds(16, 16)] = jnp.zeros((16,), _f32)
        return carry
    lax.fori_loop(0, _B, _zg, 0)

    z0 = s * _ZSTRIPE
    nz_full = _ZSTRIPE // _B
    nz_tail = _ZSTRIPE - nz_full * _B

    def _zacc(j, carry):
        pltpu.sync_copy(gbuf, acc.at[pl.ds(z0 + j * _B, _B)])
        return carry
    lax.fori_loop(0, nz_full, _zacc, 0)
    if nz_tail:
        pltpu.sync_copy(gbuf.at[pl.ds(0, nz_tail)],
                        acc.at[pl.ds(z0 + nz_full * _B, nz_tail)])
    plsc.subcore_barrier()

    base0 = s * _EPW

    def _chunk(i, carry):
        base = base0 + i * _B
        pltpu.sync_copy(row_hbm.at[pl.ds(base, _B)], irow)
        pltpu.sync_copy(col_hbm.at[pl.ds(base, _B)], icol)
        pltpu.async_copy(y_hbm.at[irow], gbuf, sem).wait()
        for j in range(_B // 16):
            sl = pl.ds(j * 16, 16)
            icol_l[sl] = _local_idx(icol[sl], c_off)
        pltpu.sync_copy(gbuf, acc.at[icol_l], add=True)
        return carry
    lax.fori_loop(0, _NFULL, _chunk, 0)

    if _TAIL:
        base = base0 + _NFULL * _B
        pltpu.sync_copy(row_hbm.at[pl.ds(base, _TAIL)], irow32)
        pltpu.sync_copy(col_hbm.at[pl.ds(base, _TAIL)], icol32)
        pltpu.async_copy(y_hbm.at[irow32], gbuf32, sem).wait()
        for j in range(_TAIL // 16):
            sl = pl.ds(j * 16, 16)
            icol_l32[sl] = _local_idx(icol32[sl], c_off)
        pltpu.sync_copy(gbuf32, acc.at[icol_l32], add=True)

    plsc.subcore_barrier()


    def _wblk(t, carry):
        cid = s + t * _NS

        @pl.when(cid < _NCHUNKSP)
        def _():
            r = cid * _WBP
            g = c_off + r
            pltpu.sync_copy(acc.at[pl.ds(r, _WBP)], obuf)
            pltpu.sync_copy(scale_hbm.at[pl.ds(g, _WBP)], sbuf)

            def _row(i, cc):
                lo = pl.ds(0, 16)
                hi = pl.ds(16, 16)
                obuf[i, lo] = obuf[i, lo] * sbuf[i, lo]
                obuf[i, hi] = obuf[i, hi] * sbuf[i, hi]
                return cc
            lax.fori_loop(0, _WBP, _row, 0)

            pltpu.sync_copy(obuf, out_hbm.at[pl.ds(g, _WBP)])
        return carry
    lax.fori_loop(0, _WROUNDSP, _wblk, 0)


_prop_kernel = functools.partial(
    pl.kernel,
    out_type=jax.ShapeDtypeStruct((_N, _K), _f32),
    mesh=_mesh,
    compiler_params=_params,
    scratch_types=[
        pltpu.VMEM_SHARED((_ACC_ROWS, _K), _f32),
        pltpu.VMEM((_B,), _i32),
        pltpu.VMEM((_B,), _i32),
        pltpu.VMEM((_B,), _i32),
        pltpu.VMEM((_TAIL,), _i32),
        pltpu.VMEM((_TAIL,), _i32),
        pltpu.VMEM((_TAIL,), _i32),
        pltpu.VMEM((_B, _K), _f32),
        pltpu.VMEM((_TAIL, _K), _f32),
        pltpu.VMEM((_WBP, _K), _f32),
        pltpu.VMEM((_WBP, _K), _f32),
        pltpu.SemaphoreType.DMA,
    ],
)(_prop_body)


def kernel(edge_index, edge_index_intents, Gu, Gi):
    row = edge_index[0]
    col = edge_index[1]
    x0 = jnp.concatenate([Gu, Gi], axis=0)
    ints = [edge_index_intents[k] for k in range(_NINT)]
    dis1, dis2, y0 = _deg_kernel(row, col, *ints, x0)
    y1 = _prop_kernel(y0, dis2, row, col)
    out = _prop_kernel(y1, dis1, row, col)
    return out.reshape(_N, _NINT, _K // _NINT)


# speedup vs baseline: 70.4430x; 1.2925x over previous
import functools

import jax
import jax.numpy as jnp
from jax import lax
from jax.experimental import pallas as pl
from jax.experimental.pallas import tpu as pltpu
from jax.experimental.pallas import tpu_sc as plsc

_N_USERS = 50000
_N_ITEMS = 50000
_N = _N_USERS + _N_ITEMS
_K = 32
_NINT = 4
_E = 1600000

_NC = 2
_NS = 16
_HALF = _N // _NC
_DUMP = _HALF
_ACC_ROWS = _HALF + 48
_ZSTRIPE = _ACC_ROWS // _NS
_B = 128
_SUP = 512
_NSB = _SUP // _B
_EPW = _E // _NS
_NSUP = _EPW // _SUP
_REM = _EPW - _NSUP * _SUP
_NFULL = _REM // _B
_TAIL = _REM - _NFULL * _B
_WB = 400
_NCHUNKS = _HALF // _WB
_WROUNDS = (_NCHUNKS + _NS - 1) // _NS
_WBP = 80
_NCHUNKSP = _HALF // _WBP
_WROUNDSP = (_NCHUNKSP + _NS - 1) // _NS

_mesh = plsc.VectorSubcoreMesh(
    core_axis_name="c", subcore_axis_name="s", num_cores=_NC, num_subcores=_NS
)
_params = pltpu.CompilerParams(
    use_tc_tiling_on_sc=False, needs_layout_passes=False
)

_f32 = jnp.float32
_i32 = jnp.int32


def _iota16():
    return lax.iota(_i32, 16)


def _rsqrt_guarded(d):
    s = jnp.maximum(d, 1.0)
    for _ in range(18):
        s = 0.5 * (s + d / s)
    return jnp.where(d > 0.0, 1.0 / s, 0.0)


def _local_idx(raw, c_off):
    v = raw - c_off
    ok = (v >= 0) & (v < _HALF)
    return jnp.where(ok, v, _DUMP)


def _deg_body(row_hbm, col_hbm, int0_hbm, int1_hbm, int2_hbm, int3_hbm, x0_hbm,
              dis1_hbm, dis2_hbm, y0_hbm,
              acc0, acc1, acc2, acc3,
              i4buf, wbuf, wbuf4, wtail, zbuf,
              irows, icols, irow_l2, icol_l2,
              irow, icol, irow_l, icol_l,
              irow32, icol32, irow_l32, icol_l32,
              dkbuf, x0buf, d1buf, d2buf, ybuf, tmp64, sem):
    c = lax.axis_index("c")
    s = lax.axis_index("s")
    c_off = c * _HALF
    iota = _iota16()
    int_hbms = (int0_hbm, int1_hbm, int2_hbm, int3_hbm)
    accs = (acc0, acc1, acc2, acc3)

    def _zw(i, carry):
        zbuf[pl.ds(i * 16, 16)] = jnp.zeros((16,), _f32)
        return carry
    lax.fori_loop(0, _B // 16, _zw, 0)

    z0 = s * _ZSTRIPE
    nz_full = _ZSTRIPE // _B
    nz_tail = _ZSTRIPE - nz_full * _B

    def _zacc(j, carry):
        for k in range(_NINT):
            pltpu.sync_copy(zbuf, accs[k].at[pl.ds(z0 + j * _B, _B)])
        return carry
    lax.fori_loop(0, nz_full, _zacc, 0)
    if nz_tail:
        for k in range(_NINT):
            pltpu.sync_copy(zbuf.at[pl.ds(0, nz_tail)],
                            accs[k].at[pl.ds(z0 + nz_full * _B, nz_tail)])
    plsc.subcore_barrier()

    base0 = s * _EPW

    def _softmax_to(dst_row, dst_off, h):
        sl = pl.ds(h * 16, 16)
        a = [i4buf[k, sl] for k in range(_NINT)]
        m = jnp.maximum(jnp.maximum(a[0], a[1]), jnp.maximum(a[2], a[3]))
        e = [jnp.exp(ak - m) for ak in a]
        ssum = (e[0] + e[1]) + (e[2] + e[3])
        for k in range(_NINT):
            wbuf[k * _NSB + dst_row, pl.ds(dst_off, 16)] = e[k] / ssum

    def _super(i, carry):
        base = base0 + i * _SUP
        pltpu.sync_copy(row_hbm.at[pl.ds(base, _SUP)], irows)
        pltpu.sync_copy(col_hbm.at[pl.ds(base, _SUP)], icols)
        for k in range(_NINT):
            pltpu.sync_copy(int_hbms[k].at[pl.ds(base, _SUP)], i4buf.at[k])
        for h in range(_SUP // 16):
            _softmax_to(h // 8, (h % 8) * 16, h)
        for j in range(_NSB):
            for h in range(_B // 16):
                sl = pl.ds(j * _B + h * 16, 16)
                d16 = pl.ds(h * 16, 16)
                irow_l2[j, d16] = _local_idx(irows[sl], c_off)
                icol_l2[j, d16] = _local_idx(icols[sl], c_off)
        started = []
        for j in range(_NSB):
            for k in range(_NINT):
                src = wbuf.at[k * _NSB + j]
                started.append(pltpu.async_copy(
                    src, accs[k].at[irow_l2.at[j]], sem, add=True))
                started.append(pltpu.async_copy(
                    src, accs[k].at[icol_l2.at[j]], sem, add=True))
        for d in started:
            d.wait()
        return carry
    lax.fori_loop(0, _NSUP, _super, 0)

    if _NFULL:
        base = base0 + _NSUP * _SUP
        pltpu.sync_copy(row_hbm.at[pl.ds(base, _B)], irow)
        pltpu.sync_copy(col_hbm.at[pl.ds(base, _B)], icol)
        for k in range(_NINT):
            pltpu.sync_copy(int_hbms[k].at[pl.ds(base, _B)], i4buf.at[k, pl.ds(0, _B)])
        for h in range(_B // 16):
            sl = pl.ds(h * 16, 16)
            a = [i4buf[k, sl] for k in range(_NINT)]
            m = jnp.maximum(jnp.maximum(a[0], a[1]), jnp.maximum(a[2], a[3]))
            e = [jnp.exp(ak - m) for ak in a]
            ssum = (e[0] + e[1]) + (e[2] + e[3])
            for k in range(_NINT):
                wbuf4[k, sl] = e[k] / ssum
        for h in range(_B // 16):
            sl = pl.ds(h * 16, 16)
            irow_l[sl] = _local_idx(irow[sl], c_off)
            icol_l[sl] = _local_idx(icol[sl], c_off)
        for k in range(_NINT):
            pltpu.sync_copy(wbuf4.at[k], accs[k].at[irow_l], add=True)
            pltpu.sync_copy(wbuf4.at[k], accs[k].at[icol_l], add=True)

    if _TAIL:
        base = base0 + _NSUP * _SUP + _NFULL * _B
        pltpu.sync_copy(row_hbm.at[pl.ds(base, _TAIL)], irow32)
        pltpu.sync_copy(col_hbm.at[pl.ds(base, _TAIL)], icol32)
        for k in range(_NINT):
            pltpu.sync_copy(int_hbms[k].at[pl.ds(base, _TAIL)],
                            i4buf.at[k, pl.ds(0, _TAIL)])
        for h in range(_TAIL // 16):
            sl = pl.ds(h * 16, 16)
            a = [i4buf[k, sl] for k in range(_NINT)]
            m = jnp.maximum(jnp.maximum(a[0], a[1]), jnp.maximum(a[2], a[3]))
            e = [jnp.exp(ak - m) for ak in a]
            ssum = (e[0] + e[1]) + (e[2] + e[3])
            for k in range(_NINT):
                wtail[k, sl] = e[k] / ssum
        for h in range(_TAIL // 16):
            sl = pl.ds(h * 16, 16)
            irow_l32[sl] = _local_idx(irow32[sl], c_off)
            icol_l32[sl] = _local_idx(icol32[sl], c_off)
        for k in range(_NINT):
            pltpu.sync_copy(wtail.at[k], accs[k].at[irow_l32], add=True)
            pltpu.sync_copy(wtail.at[k], accs[k].at[icol_l32], add=True)

    plsc.subcore_barrier()

    ibase = lax.shift_left(lax.shift_right_logical(iota, jnp.int32(3)),
                           jnp.int32(4))

    def _wblk(t, carry):
        cid = s + t * _NS

        @pl.when(cid < _NCHUNKS)
        def _():
            r = cid * _WB
            g = c_off + r
            for k in range(_NINT):
                pltpu.sync_copy(accs[k].at[pl.ds(r, _WB)], dkbuf.at[k])
            pltpu.sync_copy(x0_hbm.at[pl.ds(g, _WB)], x0buf)

            def _grp(j, cc):
                sl = pl.ds(j * 16, 16)
                for k in range(_NINT):
                    tmp64[pl.ds(k * 16, 16)] = _rsqrt_guarded(dkbuf[k, sl])
                for t16 in range(16):
                    i = j * 16 + t16
                    g0 = plsc.load_gather(tmp64, [ibase + t16])
                    g1 = plsc.load_gather(tmp64, [ibase + (32 + t16)])
                    lo = pl.ds(0, 16)
                    hi = pl.ds(16, 16)
                    d1buf[i, lo] = g0
                    d1buf[i, hi] = g1
                    d2buf[i, lo] = g0 * g0
                    d2buf[i, hi] = g1 * g1
                    ybuf[i, lo] = x0buf[i, lo] * g0
                    ybuf[i, hi] = x0buf[i, hi] * g1
                return cc
            lax.fori_loop(0, _WB // 16, _grp, 0)

            pltpu.sync_copy(d1buf, dis1_hbm.at[pl.ds(g, _WB)])
            pltpu.sync_copy(d2buf, dis2_hbm.at[pl.ds(g, _WB)])
            pltpu.sync_copy(ybuf, y0_hbm.at[pl.ds(g, _WB)])
        return carry
    lax.fori_loop(0, _WROUNDS, _wblk, 0)


_deg_kernel = functools.partial(
    pl.kernel,
    out_type=(
        jax.ShapeDtypeStruct((_N, _K), _f32),
        jax.ShapeDtypeStruct((_N, _K), _f32),
        jax.ShapeDtypeStruct((_N, _K), _f32),
    ),
    mesh=_mesh,
    compiler_params=_params,
    scratch_types=[
        pltpu.VMEM_SHARED((_ACC_ROWS,), _f32),
        pltpu.VMEM_SHARED((_ACC_ROWS,), _f32),
        pltpu.VMEM_SHARED((_ACC_ROWS,), _f32),
        pltpu.VMEM_SHARED((_ACC_ROWS,), _f32),
        pltpu.VMEM((_NINT, _SUP), _f32),
        pltpu.VMEM((_NINT * _NSB, _B), _f32),
        pltpu.VMEM((_NINT, _B), _f32),
        pltpu.VMEM((_NINT, _TAIL), _f32),
        pltpu.VMEM((_B,), _f32),
        pltpu.VMEM((_SUP,), _i32),
        pltpu.VMEM((_SUP,), _i32),
        pltpu.VMEM((_NSB, _B), _i32),
        pltpu.VMEM((_NSB, _B), _i32),
        pltpu.VMEM((_B,), _i32),
        pltpu.VMEM((_B,), _i32),
        pltpu.VMEM((_B,), _i32),
        pltpu.VMEM((_B,), _i32),
        pltpu.VMEM((_TAIL,), _i32),
        pltpu.VMEM((_TAIL,), _i32),
        pltpu.VMEM((_TAIL,), _i32),
        pltpu.VMEM((_TAIL,), _i32),
        pltpu.VMEM((_NINT, _WB), _f32),
        pltpu.VMEM((_WB, _K), _f32),
        pltpu.VMEM((_WB, _K), _f32),
        pltpu.VMEM((_WB, _K), _f32),
        pltpu.VMEM((_WB, _K), _f32),
        pltpu.VMEM((64,), _f32),
        pltpu.SemaphoreType.DMA,
    ],
)(_deg_body)


def _prop_body(y_hbm, scale_hbm, row_hbm, col_hbm, out_hbm,
               acc, irows, icols, icol_l2,
               irow, icol, icol_l, irow32, icol32, icol_l32,
               gb0, gb1, gb2, gb3, gbuf, gbuf32, obuf, sbuf, gsem, ssem):
    c = lax.axis_index("c")
    s = lax.axis_index("s")
    c_off = c * _HALF

    gbufs = (gb0, gb1, gb2, gb3)

    def _zg(i, carry):
        gbuf[i, pl.ds(0, 16)] = jnp.zeros((16,), _f32)
        gbuf[i, pl.ds(16, 16)] = jnp.zeros((16,), _f32)
        return carry
    lax.fori_loop(0, _B, _zg, 0)

    z0 = s * _ZSTRIPE
    nz_full = _ZSTRIPE // _B
    nz_tail = _ZSTRIPE - nz_full * _B

    def _zacc(j, carry):
        pltpu.sync_copy(gbuf, acc.at[pl.ds(z0 + j * _B, _B)])
        return carry
    lax.fori_loop(0, nz_full, _zacc, 0)
    if nz_tail:
        pltpu.sync_copy(gbuf.at[pl.ds(0, nz_tail)],
                        acc.at[pl.ds(z0 + nz_full * _B, nz_tail)])
    plsc.subcore_barrier()

    base0 = s * _EPW

    def _super(i, carry):
        base = base0 + i * _SUP
        pltpu.sync_copy(row_hbm.at[pl.ds(base, _SUP)], irows)
        pltpu.sync_copy(col_hbm.at[pl.ds(base, _SUP)], icols)
        gds = [pltpu.async_copy(
                   y_hbm.at[irows.at[pl.ds(j * _B, _B)]], gbufs[j], gsem)
               for j in range(_NSB)]
        for j in range(_NSB):
            for h in range(_B // 16):
                sl = pl.ds(j * _B + h * 16, 16)
                icol_l2[j, pl.ds(h * 16, 16)] = _local_idx(icols[sl], c_off)
        sds = []
        for j in range(_NSB):
            gds[j].wait()
            sds.append(pltpu.async_copy(
                gbufs[j], acc.at[icol_l2.at[j]], ssem, add=True))
        for d in sds:
            d.wait()
        return carry
    lax.fori_loop(0, _NSUP, _super, 0)

    if _NFULL:
        base = base0 + _NSUP * _SUP
        pltpu.sync_copy(row_hbm.at[pl.ds(base, _B)], irow)
        pltpu.sync_copy(col_hbm.at[pl.ds(base, _B)], icol)
        pltpu.async_copy(y_hbm.at[irow], gbuf, gsem).wait()
        for j in range(_B // 16):
            sl = pl.ds(j * 16, 16)
            icol_l[sl] = _local_idx(icol[sl], c_off)
        pltpu.sync_copy(gbuf, acc.at[icol_l], add=True)

    if _TAIL:
        base = base0 + _NSUP * _SUP + _NFULL * _B
        pltpu.sync_copy(row_hbm.at[pl.ds(base, _TAIL)], irow32)
        pltpu.sync_copy(col_hbm.at[pl.ds(base, _TAIL)], icol32)
        pltpu.async_copy(y_hbm.at[irow32], gbuf32, gsem).wait()
        for j in range(_TAIL // 16):
            sl = pl.ds(j * 16, 16)
            icol_l32[sl] = _local_idx(icol32[sl], c_off)
        pltpu.sync_copy(gbuf32, acc.at[icol_l32], add=True)

    plsc.subcore_barrier()


    def _wblk(t, carry):
        cid = s + t * _NS

        @pl.when(cid < _NCHUNKSP)
        def _():
            r = cid * _WBP
            g = c_off + r
            pltpu.sync_copy(acc.at[pl.ds(r, _WBP)], obuf)
            pltpu.sync_copy(scale_hbm.at[pl.ds(g, _WBP)], sbuf)

            def _row(i, cc):
                lo = pl.ds(0, 16)
                hi = pl.ds(16, 16)
                obuf[i, lo] = obuf[i, lo] * sbuf[i, lo]
                obuf[i, hi] = obuf[i, hi] * sbuf[i, hi]
                return cc
            lax.fori_loop(0, _WBP, _row, 0)

            pltpu.sync_copy(obuf, out_hbm.at[pl.ds(g, _WBP)])
        return carry
    lax.fori_loop(0, _WROUNDSP, _wblk, 0)


_prop_kernel = functools.partial(
    pl.kernel,
    out_type=jax.ShapeDtypeStruct((_N, _K), _f32),
    mesh=_mesh,
    compiler_params=_params,
    scratch_types=[
        pltpu.VMEM_SHARED((_ACC_ROWS, _K), _f32),
        pltpu.VMEM((_SUP,), _i32),
        pltpu.VMEM((_SUP,), _i32),
        pltpu.VMEM((_NSB, _B), _i32),
        pltpu.VMEM((_B,), _i32),
        pltpu.VMEM((_B,), _i32),
        pltpu.VMEM((_B,), _i32),
        pltpu.VMEM((_TAIL,), _i32),
        pltpu.VMEM((_TAIL,), _i32),
        pltpu.VMEM((_TAIL,), _i32),
        pltpu.VMEM((_B, _K), _f32),
        pltpu.VMEM((_B, _K), _f32),
        pltpu.VMEM((_B, _K), _f32),
        pltpu.VMEM((_B, _K), _f32),
        pltpu.VMEM((_B, _K), _f32),
        pltpu.VMEM((_TAIL, _K), _f32),
        pltpu.VMEM((_WBP, _K), _f32),
        pltpu.VMEM((_WBP, _K), _f32),
        pltpu.SemaphoreType.DMA,
        pltpu.SemaphoreType.DMA,
    ],
)(_prop_body)


def kernel(edge_index, edge_index_intents, Gu, Gi):
    row = edge_index[0]
    col = edge_index[1]
    x0 = jnp.concatenate([Gu, Gi], axis=0)
    ints = [edge_index_intents[k] for k in range(_NINT)]
    dis1, dis2, y0 = _deg_kernel(row, col, *ints, x0)
    y1 = _prop_kernel(y0, dis2, row, col)
    out = _prop_kernel(y1, dis1, row, col)
    return out.reshape(_N, _NINT, _K // _NINT)


# speedup vs baseline: 70.8481x; 1.0058x over previous
import functools

import jax
import jax.numpy as jnp
from jax import lax
from jax.experimental import pallas as pl
from jax.experimental.pallas import tpu as pltpu
from jax.experimental.pallas import tpu_sc as plsc

_N_USERS = 50000
_N_ITEMS = 50000
_N = _N_USERS + _N_ITEMS
_K = 32
_NINT = 4
_E = 1600000

_NC = 2
_NS = 16
_HALF = _N // _NC
_DUMP = _HALF
_ACC_ROWS = _HALF + 48
_ZSTRIPE = _ACC_ROWS // _NS
_EPW = _E // _NS

_SUPK = 1024
_NSUPK = _EPW // _SUPK
_REMK = _EPW - _NSUPK * _SUPK
_B = 128
_NFULLK = _REMK // _B
_TAIL = _REMK - _NFULLK * _B

_SUP = 512
_NSUP = _EPW // _SUP
_REMP = _EPW - _NSUP * _SUP

_WB = 400
_NCHUNKS = _HALF // _WB
_WROUNDS = (_NCHUNKS + _NS - 1) // _NS
_WBP = 80
_NCHUNKSP = _HALF // _WBP
_WROUNDSP = (_NCHUNKSP + _NS - 1) // _NS

_mesh = plsc.VectorSubcoreMesh(
    core_axis_name="c", subcore_axis_name="s", num_cores=_NC, num_subcores=_NS
)
_params = pltpu.CompilerParams(
    use_tc_tiling_on_sc=False, needs_layout_passes=False
)

_f32 = jnp.float32
_i32 = jnp.int32


def _iota16():
    return lax.iota(_i32, 16)


def _rsqrt_guarded(d):
    s = jnp.maximum(d, 1.0)
    for _ in range(18):
        s = 0.5 * (s + d / s)
    return jnp.where(d > 0.0, 1.0 / s, 0.0)


def _local_idx(raw, c_off):
    v = raw - c_off
    ok = (v >= 0) & (v < _HALF)
    return jnp.where(ok, v, _DUMP)


def _deg_body(row_hbm, col_hbm, int0_hbm, int1_hbm, int2_hbm, int3_hbm, x0_hbm,
              dis1_hbm, dis2_hbm, y0_hbm,
              acc0, acc1, acc2, acc3,
              i4buf, wbuf4, zbuf,
              irow, icol, irow_l, icol_l,
              dkbuf, x0buf, d1buf, d2buf, ybuf, tmp64, sem):
    c = lax.axis_index("c")
    s = lax.axis_index("s")
    c_off = c * _HALF
    iota = _iota16()
    int_hbms = (int0_hbm, int1_hbm, int2_hbm, int3_hbm)
    accs = (acc0, acc1, acc2, acc3)

    def _zw(i, carry):
        zbuf[pl.ds(i * 16, 16)] = jnp.zeros((16,), _f32)
        return carry
    lax.fori_loop(0, _SUPK // 16, _zw, 0)

    z0 = s * _ZSTRIPE
    nz_full = _ZSTRIPE // _SUPK
    nz_tail = _ZSTRIPE - nz_full * _SUPK

    def _zacc(j, carry):
        for k in range(_NINT):
            pltpu.sync_copy(zbuf, accs[k].at[pl.ds(z0 + j * _SUPK, _SUPK)])
        return carry
    lax.fori_loop(0, nz_full, _zacc, 0)
    if nz_tail:
        for k in range(_NINT):
            pltpu.sync_copy(zbuf.at[pl.ds(0, nz_tail)],
                            accs[k].at[pl.ds(z0 + nz_full * _SUPK, nz_tail)])
    plsc.subcore_barrier()

    base0 = s * _EPW

    def _softmax(h):
        sl = pl.ds(h * 16, 16)
        a = [i4buf[k, sl] for k in range(_NINT)]
        m = jnp.maximum(jnp.maximum(a[0], a[1]), jnp.maximum(a[2], a[3]))
        e = [jnp.exp(ak - m) for ak in a]
        ssum = (e[0] + e[1]) + (e[2] + e[3])
        for k in range(_NINT):
            wbuf4[k, sl] = e[k] / ssum

    def _edges(base, n, ir, ic, irl, icl):
        pltpu.sync_copy(row_hbm.at[pl.ds(base, n)], ir)
        pltpu.sync_copy(col_hbm.at[pl.ds(base, n)], ic)
        for k in range(_NINT):
            pltpu.sync_copy(int_hbms[k].at[pl.ds(base, n)],
                            i4buf.at[k, pl.ds(0, n)] if n != _SUPK
                            else i4buf.at[k])
        for h in range(n // 16):
            _softmax(h)
        for h in range(n // 16):
            sl = pl.ds(h * 16, 16)
            irl[sl] = _local_idx(ir[sl], c_off)
            icl[sl] = _local_idx(ic[sl], c_off)
        started = []
        for k in range(_NINT):
            src = wbuf4.at[k] if n == _SUPK else wbuf4.at[k, pl.ds(0, n)]
            started.append(pltpu.async_copy(
                src, accs[k].at[irl], sem, add=True))
            started.append(pltpu.async_copy(
                src, accs[k].at[icl], sem, add=True))
        for d in started:
            d.wait()

    def _super(i, carry):
        _edges(base0 + i * _SUPK, _SUPK, irow, icol, irow_l, icol_l)
        return carry
    lax.fori_loop(0, _NSUPK, _super, 0)

    rem0 = base0 + _NSUPK * _SUPK
    for q in range(_NFULLK):
        _edges(rem0 + q * _B, _B,
               irow.at[pl.ds(0, _B)], icol.at[pl.ds(0, _B)],
               irow_l.at[pl.ds(0, _B)], icol_l.at[pl.ds(0, _B)])
    if _TAIL:
        _edges(rem0 + _NFULLK * _B, _TAIL,
               irow.at[pl.ds(0, _TAIL)], icol.at[pl.ds(0, _TAIL)],
               irow_l.at[pl.ds(0, _TAIL)], icol_l.at[pl.ds(0, _TAIL)])

    plsc.subcore_barrier()

    ibase = lax.shift_left(lax.shift_right_logical(iota, jnp.int32(3)),
                           jnp.int32(4))

    def _wblk(t, carry):
        cid = s + t * _NS

        @pl.when(cid < _NCHUNKS)
        def _():
            r = cid * _WB
            g = c_off + r
            for k in range(_NINT):
                pltpu.sync_copy(accs[k].at[pl.ds(r, _WB)], dkbuf.at[k])
            pltpu.sync_copy(x0_hbm.at[pl.ds(g, _WB)], x0buf)

            def _grp(j, cc):
                sl = pl.ds(j * 16, 16)
                for k in range(_NINT):
                    tmp64[pl.ds(k * 16, 16)] = _rsqrt_guarded(dkbuf[k, sl])
                for t16 in range(16):
                    i = j * 16 + t16
                    g0 = plsc.load_gather(tmp64, [ibase + t16])
                    g1 = plsc.load_gather(tmp64, [ibase + (32 + t16)])
                    lo = pl.ds(0, 16)
                    hi = pl.ds(16, 16)
                    d1buf[i, lo] = g0
                    d1buf[i, hi] = g1
                    d2buf[i, lo] = g0 * g0
                    d2buf[i, hi] = g1 * g1
                    ybuf[i, lo] = x0buf[i, lo] * g0
                    ybuf[i, hi] = x0buf[i, hi] * g1
                return cc
            lax.fori_loop(0, _WB // 16, _grp, 0)

            pltpu.sync_copy(d1buf, dis1_hbm.at[pl.ds(g, _WB)])
            pltpu.sync_copy(d2buf, dis2_hbm.at[pl.ds(g, _WB)])
            pltpu.sync_copy(ybuf, y0_hbm.at[pl.ds(g, _WB)])
        return carry
    lax.fori_loop(0, _WROUNDS, _wblk, 0)


_deg_kernel = functools.partial(
    pl.kernel,
    out_type=(
        jax.ShapeDtypeStruct((_N, _K), _f32),
        jax.ShapeDtypeStruct((_N, _K), _f32),
        jax.ShapeDtypeStruct((_N, _K), _f32),
    ),
    mesh=_mesh,
    compiler_params=_params,
    scratch_types=[
        pltpu.VMEM_SHARED((_ACC_ROWS,), _f32),
        pltpu.VMEM_SHARED((_ACC_ROWS,), _f32),
        pltpu.VMEM_SHARED((_ACC_ROWS,), _f32),
        pltpu.VMEM_SHARED((_ACC_ROWS,), _f32),
        pltpu.VMEM((_NINT, _SUPK), _f32),
        pltpu.VMEM((_NINT, _SUPK), _f32),
        pltpu.VMEM((_SUPK,), _f32),
        pltpu.VMEM((_SUPK,), _i32),
        pltpu.VMEM((_SUPK,), _i32),
        pltpu.VMEM((_SUPK,), _i32),
        pltpu.VMEM((_SUPK,), _i32),
        pltpu.VMEM((_NINT, _WB), _f32),
        pltpu.VMEM((_WB, _K), _f32),
        pltpu.VMEM((_WB, _K), _f32),
        pltpu.VMEM((_WB, _K), _f32),
        pltpu.VMEM((_WB, _K), _f32),
        pltpu.VMEM((64,), _f32),
        pltpu.SemaphoreType.DMA,
    ],
)(_deg_body)


def _prop_body(y_hbm, scale_hbm, row_hbm, col_hbm, out_hbm,
               acc, irows, icols, icol_l,
               gbig, obuf, sbuf, gsem, ssem):
    c = lax.axis_index("c")
    s = lax.axis_index("s")
    c_off = c * _HALF

    def _zg(i, carry):
        gbig[i, pl.ds(0, 16)] = jnp.zeros((16,), _f32)
        gbig[i, pl.ds(16, 16)] = jnp.zeros((16,), _f32)
        return carry
    lax.fori_loop(0, _SUP, _zg, 0)

    z0 = s * _ZSTRIPE
    nz_full = _ZSTRIPE // _SUP
    nz_tail = _ZSTRIPE - nz_full * _SUP

    def _zacc(j, carry):
        pltpu.sync_copy(gbig, acc.at[pl.ds(z0 + j * _SUP, _SUP)])
        return carry
    lax.fori_loop(0, nz_full, _zacc, 0)
    if nz_tail:
        pltpu.sync_copy(gbig.at[pl.ds(0, nz_tail)],
                        acc.at[pl.ds(z0 + nz_full * _SUP, nz_tail)])
    plsc.subcore_barrier()

    base0 = s * _EPW

    def _edges(base, n, ir, icl, gb):
        pltpu.sync_copy(row_hbm.at[pl.ds(base, n)], ir)
        pltpu.sync_copy(col_hbm.at[pl.ds(base, n)], icols.at[pl.ds(0, n)])
        gd = pltpu.async_copy(y_hbm.at[ir], gb, gsem)
        for h in range(n // 16):
            sl = pl.ds(h * 16, 16)
            icl[sl] = _local_idx(icols[sl], c_off)
        gd.wait()
        pltpu.async_copy(gb, acc.at[icl], ssem, add=True).wait()

    def _super(i, carry):
        _edges(base0 + i * _SUP, _SUP, irows, icol_l, gbig)
        return carry
    lax.fori_loop(0, _NSUP, _super, 0)

    rem0 = base0 + _NSUP * _SUP
    _edges(rem0, _B, irows.at[pl.ds(0, _B)], icol_l.at[pl.ds(0, _B)],
           gbig.at[pl.ds(0, _B)])
    _edges(rem0 + _B, _TAIL, irows.at[pl.ds(0, _TAIL)],
           icol_l.at[pl.ds(0, _TAIL)], gbig.at[pl.ds(0, _TAIL)])

    plsc.subcore_barrier()


    def _wblk(t, carry):
        cid = s + t * _NS

        @pl.when(cid < _NCHUNKSP)
        def _():
            r = cid * _WBP
            g = c_off + r
            pltpu.sync_copy(acc.at[pl.ds(r, _WBP)], obuf)
            pltpu.sync_copy(scale_hbm.at[pl.ds(g, _WBP)], sbuf)

            def _row(i, cc):
                lo = pl.ds(0, 16)
                hi = pl.ds(16, 16)
                obuf[i, lo] = obuf[i, lo] * sbuf[i, lo]
                obuf[i, hi] = obuf[i, hi] * sbuf[i, hi]
                return cc
            lax.fori_loop(0, _WBP, _row, 0)

            pltpu.sync_copy(obuf, out_hbm.at[pl.ds(g, _WBP)])
        return carry
    lax.fori_loop(0, _WROUNDSP, _wblk, 0)


_prop_kernel = functools.partial(
    pl.kernel,
    out_type=jax.ShapeDtypeStruct((_N, _K), _f32),
    mesh=_mesh,
    compiler_params=_params,
    scratch_types=[
        pltpu.VMEM_SHARED((_ACC_ROWS, _K), _f32),
        pltpu.VMEM((_SUP,), _i32),
        pltpu.VMEM((_SUP,), _i32),
        pltpu.VMEM((_SUP,), _i32),
        pltpu.VMEM((_SUP, _K), _f32),
        pltpu.VMEM((_WBP, _K), _f32),
        pltpu.VMEM((_WBP, _K), _f32),
        pltpu.SemaphoreType.DMA,
        pltpu.SemaphoreType.DMA,
    ],
)(_prop_body)


def kernel(edge_index, edge_index_intents, Gu, Gi):
    row = edge_index[0]
    col = edge_index[1]
    x0 = jnp.concatenate([Gu, Gi], axis=0)
    ints = [edge_index_intents[k] for k in range(_NINT)]
    dis1, dis2, y0 = _deg_kernel(row, col, *ints, x0)
    y1 = _prop_kernel(y0, dis2, row, col)
    out = _prop_kernel(y1, dis1, row, col)
    return out.reshape(_N, _NINT, _K // _NINT)


# speedup vs baseline: 79.4331x; 1.1212x over previous
import functools

import jax
import jax.numpy as jnp
from jax import lax
from jax.experimental import pallas as pl
from jax.experimental.pallas import tpu as pltpu
from jax.experimental.pallas import tpu_sc as plsc

_N_USERS = 50000
_N_ITEMS = 50000
_N = _N_USERS + _N_ITEMS
_K = 32
_NINT = 4
_E = 1600000

_NC = 2
_NS = 16
_HALF = _N // _NC
_DUMP = _HALF
_ACC_ROWS = _HALF + 48
_ZSTRIPE = _ACC_ROWS // _NS
_EPW = _E // _NS

_SUPK = 1024
_NSUPK = _EPW // _SUPK
_REMK = _EPW - _NSUPK * _SUPK
_B = 128
_NFULLK = _REMK // _B
_TAIL = _REMK - _NFULLK * _B

_SUP = 512
_NSUP = _EPW // _SUP
_REMP = _EPW - _NSUP * _SUP

_WB = 80
_NCHUNKS = _HALF // _WB
_WROUNDS = (_NCHUNKS + _NS - 1) // _NS
_WBP = 80
_NCHUNKSP = _HALF // _WBP
_WROUNDSP = (_NCHUNKSP + _NS - 1) // _NS

_mesh = plsc.VectorSubcoreMesh(
    core_axis_name="c", subcore_axis_name="s", num_cores=_NC, num_subcores=_NS
)
_params = pltpu.CompilerParams(
    use_tc_tiling_on_sc=False, needs_layout_passes=False
)

_f32 = jnp.float32
_i32 = jnp.int32


def _iota16():
    return lax.iota(_i32, 16)


def _rsqrt_guarded(d):
    s = jnp.maximum(d, 1.0)
    for _ in range(18):
        s = 0.5 * (s + d / s)
    return jnp.where(d > 0.0, 1.0 / s, 0.0)


def _local_idx(raw, c_off):
    v = raw - c_off
    ok = (v >= 0) & (v < _HALF)
    return jnp.where(ok, v, _DUMP)


def _deg_body(row_hbm, col_hbm, int0_hbm, int1_hbm, int2_hbm, int3_hbm, x0_hbm,
              dis1_hbm, dis2_hbm, y0_hbm,
              accd,
              i4buf, wbuf_t,
              irow, icol, irow_l, icol_l,
              dbuf, x0buf, d1buf, d2buf, ybuf, tmp16, sem):
    c = lax.axis_index("c")
    s = lax.axis_index("s")
    c_off = c * _HALF
    iota = _iota16()
    int_hbms = (int0_hbm, int1_hbm, int2_hbm, int3_hbm)
    kcol = [jnp.full((16,), k, _i32) for k in range(_NINT)]

    z16 = jnp.zeros((16,), _f32)

    def _zw(i, carry):
        wbuf_t[i, pl.ds(0, 16)] = z16
        return carry
    lax.fori_loop(0, _SUPK, _zw, 0)

    z0 = s * _ZSTRIPE
    nz_full = _ZSTRIPE // _SUPK
    nz_tail = _ZSTRIPE - nz_full * _SUPK

    def _zacc(j, carry):
        pltpu.sync_copy(wbuf_t, accd.at[pl.ds(z0 + j * _SUPK, _SUPK)])
        return carry
    lax.fori_loop(0, nz_full, _zacc, 0)
    if nz_tail:
        pltpu.sync_copy(wbuf_t.at[pl.ds(0, nz_tail)],
                        accd.at[pl.ds(z0 + nz_full * _SUPK, nz_tail)])
    plsc.subcore_barrier()

    base0 = s * _EPW

    def _softmax(h):
        sl = pl.ds(h * 16, 16)
        a = [i4buf[k, sl] for k in range(_NINT)]
        m = jnp.maximum(jnp.maximum(a[0], a[1]), jnp.maximum(a[2], a[3]))
        e = [jnp.exp(ak - m) for ak in a]
        ssum = (e[0] + e[1]) + (e[2] + e[3])
        rows = jnp.int32(h * 16) + iota
        for k in range(_NINT):
            plsc.store_scatter(wbuf_t, [rows, kcol[k]], e[k] / ssum)

    def _edges(base, n, ir, ic, irl, icl):
        pltpu.sync_copy(row_hbm.at[pl.ds(base, n)], ir)
        pltpu.sync_copy(col_hbm.at[pl.ds(base, n)], ic)
        for k in range(_NINT):
            pltpu.sync_copy(int_hbms[k].at[pl.ds(base, n)],
                            i4buf.at[k, pl.ds(0, n)] if n != _SUPK
                            else i4buf.at[k])
        for h in range(n // 16):
            _softmax(h)
        for h in range(n // 16):
            sl = pl.ds(h * 16, 16)
            irl[sl] = _local_idx(ir[sl], c_off)
            icl[sl] = _local_idx(ic[sl], c_off)
        src = wbuf_t if n == _SUPK else wbuf_t.at[pl.ds(0, n)]
        d0 = pltpu.async_copy(src, accd.at[irl], sem, add=True)
        d1 = pltpu.async_copy(src, accd.at[icl], sem, add=True)
        d0.wait()
        d1.wait()

    def _super(i, carry):
        _edges(base0 + i * _SUPK, _SUPK, irow, icol, irow_l, icol_l)
        return carry
    lax.fori_loop(0, _NSUPK, _super, 0)

    rem0 = base0 + _NSUPK * _SUPK
    for q in range(_NFULLK):
        _edges(rem0 + q * _B, _B,
               irow.at[pl.ds(0, _B)], icol.at[pl.ds(0, _B)],
               irow_l.at[pl.ds(0, _B)], icol_l.at[pl.ds(0, _B)])
    if _TAIL:
        _edges(rem0 + _NFULLK * _B, _TAIL,
               irow.at[pl.ds(0, _TAIL)], icol.at[pl.ds(0, _TAIL)],
               irow_l.at[pl.ds(0, _TAIL)], icol_l.at[pl.ds(0, _TAIL)])

    plsc.subcore_barrier()

    idx_a = lax.shift_right_logical(iota, jnp.int32(3))

    def _wblk(t, carry):
        cid = s + t * _NS

        @pl.when(cid < _NCHUNKS)
        def _():
            r = cid * _WB
            g = c_off + r
            pltpu.sync_copy(accd.at[pl.ds(r, _WB)], dbuf)
            pltpu.sync_copy(x0_hbm.at[pl.ds(g, _WB)], x0buf)

            def _row(i, cc):
                d = _rsqrt_guarded(dbuf[i, pl.ds(0, 16)])
                tmp16[...] = d
                g0 = plsc.load_gather(tmp16, [idx_a])
                g1 = plsc.load_gather(tmp16, [idx_a + 2])
                lo = pl.ds(0, 16)
                hi = pl.ds(16, 16)
                d1buf[i, lo] = g0
                d1buf[i, hi] = g1
                d2buf[i, lo] = g0 * g0
                d2buf[i, hi] = g1 * g1
                ybuf[i, lo] = x0buf[i, lo] * g0
                ybuf[i, hi] = x0buf[i, hi] * g1
                return cc
            lax.fori_loop(0, _WB, _row, 0)

            pltpu.sync_copy(d1buf, dis1_hbm.at[pl.ds(g, _WB)])
            pltpu.sync_copy(d2buf, dis2_hbm.at[pl.ds(g, _WB)])
            pltpu.sync_copy(ybuf, y0_hbm.at[pl.ds(g, _WB)])
        return carry
    lax.fori_loop(0, _WROUNDS, _wblk, 0)


_deg_kernel = functools.partial(
    pl.kernel,
    out_type=(
        jax.ShapeDtypeStruct((_N, _K), _f32),
        jax.ShapeDtypeStruct((_N, _K), _f32),
        jax.ShapeDtypeStruct((_N, _K), _f32),
    ),
    mesh=_mesh,
    compiler_params=_params,
    scratch_types=[
        pltpu.VMEM_SHARED((_ACC_ROWS, 16), _f32),
        pltpu.VMEM((_NINT, _SUPK), _f32),
        pltpu.VMEM((_SUPK, 16), _f32),
        pltpu.VMEM((_SUPK,), _i32),
        pltpu.VMEM((_SUPK,), _i32),
        pltpu.VMEM((_SUPK,), _i32),
        pltpu.VMEM((_SUPK,), _i32),
        pltpu.VMEM((_WB, 16), _f32),
        pltpu.VMEM((_WB, _K), _f32),
        pltpu.VMEM((_WB, _K), _f32),
        pltpu.VMEM((_WB, _K), _f32),
        pltpu.VMEM((_WB, _K), _f32),
        pltpu.VMEM((16,), _f32),
        pltpu.SemaphoreType.DMA,
    ],
)(_deg_body)


def _prop_body(y_hbm, scale_hbm, row_hbm, col_hbm, out_hbm,
               acc, irows, icols, icol_l,
               gbig, obuf, sbuf, gsem, ssem):
    c = lax.axis_index("c")
    s = lax.axis_index("s")
    c_off = c * _HALF

    def _zg(i, carry):
        gbig[i, pl.ds(0, 16)] = jnp.zeros((16,), _f32)
        gbig[i, pl.ds(16, 16)] = jnp.zeros((16,), _f32)
        return carry
    lax.fori_loop(0, _SUP, _zg, 0)

    z0 = s * _ZSTRIPE
    nz_full = _ZSTRIPE // _SUP
    nz_tail = _ZSTRIPE - nz_full * _SUP

    def _zacc(j, carry):
        pltpu.sync_copy(gbig, acc.at[pl.ds(z0 + j * _SUP, _SUP)])
        return carry
    lax.fori_loop(0, nz_full, _zacc, 0)
    if nz_tail:
        pltpu.sync_copy(gbig.at[pl.ds(0, nz_tail)],
                        acc.at[pl.ds(z0 + nz_full * _SUP, nz_tail)])
    plsc.subcore_barrier()

    base0 = s * _EPW

    def _edges(base, n, ir, icl, gb):
        pltpu.sync_copy(row_hbm.at[pl.ds(base, n)], ir)
        pltpu.sync_copy(col_hbm.at[pl.ds(base, n)], icols.at[pl.ds(0, n)])
        gd = pltpu.async_copy(y_hbm.at[ir], gb, gsem)
        for h in range(n // 16):
            sl = pl.ds(h * 16, 16)
            icl[sl] = _local_idx(icols[sl], c_off)
        gd.wait()
        pltpu.async_copy(gb, acc.at[icl], ssem, add=True).wait()

    def _super(i, carry):
        _edges(base0 + i * _SUP, _SUP, irows, icol_l, gbig)
        return carry
    lax.fori_loop(0, _NSUP, _super, 0)

    rem0 = base0 + _NSUP * _SUP
    _edges(rem0, _B, irows.at[pl.ds(0, _B)], icol_l.at[pl.ds(0, _B)],
           gbig.at[pl.ds(0, _B)])
    _edges(rem0 + _B, _TAIL, irows.at[pl.ds(0, _TAIL)],
           icol_l.at[pl.ds(0, _TAIL)], gbig.at[pl.ds(0, _TAIL)])

    plsc.subcore_barrier()


    def _wblk(t, carry):
        cid = s + t * _NS

        @pl.when(cid < _NCHUNKSP)
        def _():
            r = cid * _WBP
            g = c_off + r
            pltpu.sync_copy(acc.at[pl.ds(r, _WBP)], obuf)
            pltpu.sync_copy(scale_hbm.at[pl.ds(g, _WBP)], sbuf)

            def _row(i, cc):
                lo = pl.ds(0, 16)
                hi = pl.ds(16, 16)
                obuf[i, lo] = obuf[i, lo] * sbuf[i, lo]
                obuf[i, hi] = obuf[i, hi] * sbuf[i, hi]
                return cc
            lax.fori_loop(0, _WBP, _row, 0)

            pltpu.sync_copy(obuf, out_hbm.at[pl.ds(g, _WBP)])
        return carry
    lax.fori_loop(0, _WROUNDSP, _wblk, 0)


_prop_kernel = functools.partial(
    pl.kernel,
    out_type=jax.ShapeDtypeStruct((_N, _K), _f32),
    mesh=_mesh,
    compiler_params=_params,
    scratch_types=[
        pltpu.VMEM_SHARED((_ACC_ROWS, _K), _f32),
        pltpu.VMEM((_SUP,), _i32),
        pltpu.VMEM((_SUP,), _i32),
        pltpu.VMEM((_SUP,), _i32),
        pltpu.VMEM((_SUP, _K), _f32),
        pltpu.VMEM((_WBP, _K), _f32),
        pltpu.VMEM((_WBP, _K), _f32),
        pltpu.SemaphoreType.DMA,
        pltpu.SemaphoreType.DMA,
    ],
)(_prop_body)


def kernel(edge_index, edge_index_intents, Gu, Gi):
    row = edge_index[0]
    col = edge_index[1]
    x0 = jnp.concatenate([Gu, Gi], axis=0)
    ints = [edge_index_intents[k] for k in range(_NINT)]
    dis1, dis2, y0 = _deg_kernel(row, col, *ints, x0)
    y1 = _prop_kernel(y0, dis2, row, col)
    out = _prop_kernel(y1, dis1, row, col)
    return out.reshape(_N, _NINT, _K // _NINT)
